# baseline (speedup 1.0000x reference)
import functools

import jax
import jax.numpy as jnp
from jax import lax
from jax.experimental import pallas as pl
from jax.experimental.pallas import tpu as pltpu
from jax.experimental.pallas import tpu_sc as plsc

VOCAB = 100000
DIM = 512
BATCH = 4
SEQ = 8192
NTOK = BATCH * SEQ

NC = 2
NS = 16
NW = NC * NS
TOK_PER_W = NTOK // NW
CHUNK = 64
NCHUNK = TOK_PER_W // CHUNK
NBUF = 3
NSBUF = 2
SROWS = 16
DIRECT_EVERY = 8


def _embed_body(tokens_hbm, table_hbm, out_hbm, idx_v, *scratch):
    cid = lax.axis_index("c")
    sid = lax.axis_index("s")
    wid = sid * NC + cid
    base = wid * TOK_PER_W
    pltpu.sync_copy(tokens_hbm.at[pl.ds(base, TOK_PER_W)], idx_v)

    bufs = scratch[:NBUF]
    spm = scratch[NBUF]
    gsems = scratch[NBUF + 1:NBUF + 1 + NBUF]
    osems = scratch[NBUF + 1 + NBUF:NBUF + 1 + 2 * NBUF]
    dsems = scratch[NBUF + 1 + 2 * NBUF:]
    gat = [None] * NBUF
    out = [None] * NBUF
    dma = [None] * NSBUF
    look = NBUF - 1
    nsp = 0
    for c in range(NCHUNK + look):
        if c < NCHUNK:
            b = c % NBUF
            if out[b] is not None:
                out[b].wait()
                out[b] = None
            gat[b] = pltpu.async_copy(
                table_hbm.at[idx_v.at[pl.ds(c * CHUNK, CHUNK)]],
                bufs[b], gsems[b])
        d = c - look
        if d >= 0:
            b = d % NBUF
            gat[b].wait()
            if d % DIRECT_EVERY == DIRECT_EVERY - 1:
                out[b] = pltpu.async_copy(
                    bufs[b], out_hbm.at[pl.ds(base + d * CHUNK, CHUNK)],
                    osems[b])
            else:
                for h in range(CHUNK // SROWS):
                    s = nsp % NSBUF
                    nsp += 1
                    if dma[s] is not None:
                        dma[s].wait()
                    pltpu.sync_copy(bufs[b].at[pl.ds(h * SROWS, SROWS)],
                                    spm.at[sid, s])
                    dma[s] = pltpu.async_copy(
                        spm.at[sid, s],
                        out_hbm.at[pl.ds(base + d * CHUNK + h * SROWS, SROWS)],
                        dsems[s])
    for h in out + dma:
        if h is not None:
            h.wait()


@jax.jit
def _embed(tokens_flat, table):
    mesh = plsc.VectorSubcoreMesh(core_axis_name="c", subcore_axis_name="s")
    return pl.kernel(
        _embed_body,
        out_type=jax.ShapeDtypeStruct((NTOK, DIM), jnp.float32),
        mesh=mesh,
        scratch_types=(
            [pltpu.VMEM((TOK_PER_W,), jnp.int32)]
            + [pltpu.VMEM((CHUNK, DIM), jnp.float32)] * NBUF
            + [pltpu.VMEM_SHARED((NS, NSBUF, SROWS, DIM), jnp.float32)]
            + [pltpu.SemaphoreType.DMA] * (2 * NBUF + NSBUF)
        ),
    )(tokens_flat, table)


def kernel(tokens, tok_embeddings_weight):
    tokens_flat = tokens.reshape(-1).astype(jnp.int32)
    out = _embed(tokens_flat, tok_embeddings_weight)
    return out.reshape(BATCH, SEQ, DIM)


# speedup vs baseline: 1.0284x; 1.0284x over previous
import functools

import jax
import jax.numpy as jnp
from jax import lax
from jax.experimental import pallas as pl
from jax.experimental.pallas import tpu as pltpu
from jax.experimental.pallas import tpu_sc as plsc

VOCAB = 100000
DIM = 512
BATCH = 4
SEQ = 8192
NTOK = BATCH * SEQ

NC = 2
NS = 16
NW = NC * NS
TOK_PER_W = NTOK // NW
CHUNK = 120
CHUNK_OFFS = [0, 120, 240, 360, 480, 600, 720, 840, 960]
CHUNK_LENS = [120] * 8 + [64]
NCHUNK = len(CHUNK_OFFS)
NBUF = 2


def _embed_body(tokens_hbm, table_hbm, out_hbm, idx_v, *scratch):
    cid = lax.axis_index("c")
    sid = lax.axis_index("s")
    wid = sid * NC + cid
    base = wid * TOK_PER_W
    pltpu.sync_copy(tokens_hbm.at[pl.ds(base, TOK_PER_W)], idx_v)

    bufs = scratch[:NBUF]
    gsems = scratch[NBUF:2 * NBUF]
    osems = scratch[2 * NBUF:]
    gat = [None] * NBUF
    out = [None] * NBUF
    look = NBUF - 1
    for c in range(NCHUNK + look):
        if c < NCHUNK:
            b = c % NBUF
            if out[b] is not None:
                out[b].wait()
            n = CHUNK_LENS[c]
            gat[b] = pltpu.async_copy(
                table_hbm.at[idx_v.at[pl.ds(CHUNK_OFFS[c], n)]],
                bufs[b].at[pl.ds(0, n)], gsems[b])
        d = c - look
        if d >= 0:
            b = d % NBUF
            gat[b].wait()
            n = CHUNK_LENS[d]
            out[b] = pltpu.async_copy(
                bufs[b].at[pl.ds(0, n)],
                out_hbm.at[pl.ds(base + CHUNK_OFFS[d], n)],
                osems[b])
    for b in range(NBUF):
        if out[b] is not None:
            out[b].wait()


@jax.jit
def _embed(tokens_flat, table):
    mesh = plsc.VectorSubcoreMesh(core_axis_name="c", subcore_axis_name="s")
    return pl.kernel(
        _embed_body,
        out_type=jax.ShapeDtypeStruct((NTOK, DIM), jnp.float32),
        mesh=mesh,
        scratch_types=(
            [pltpu.VMEM((TOK_PER_W,), jnp.int32)]
            + [pltpu.VMEM((CHUNK, DIM), jnp.float32)] * NBUF
            + [pltpu.SemaphoreType.DMA] * (2 * NBUF)
        ),
    )(tokens_flat, table)


def kernel(tokens, tok_embeddings_weight):
    tokens_flat = tokens.reshape(-1).astype(jnp.int32)
    out = _embed(tokens_flat, tok_embeddings_weight)
    return out.reshape(BATCH, SEQ, DIM)
